# Initial kernel scaffold; baseline (speedup 1.0000x reference)
import functools

import jax
import jax.numpy as jnp
from jax import lax
from jax.experimental import pallas as pl
from jax.experimental.pallas import tpu as pltpu
from jax.experimental.pallas import tpu_sc as plsc

N = 10000
E = 320000
DIM = 128
NC = 2
NS = 16
NW = NC * NS
LANES = 16

NPAD = 10112
NSLICE = NPAD // NS
ROWS_PER_W = 80
EROWS = NW * ROWS_PER_W
EPAD = EROWS * 128

A_CHUNK = 8
B_CHUNK = 2

_mesh = plsc.VectorSubcoreMesh(
    core_axis_name="c", subcore_axis_name="s", num_cores=NC, num_subcores=NS)


BLK1 = 632

def _project_body(h_ref, wt_ref, b_ref, wa_ref, ba_ref, z_ref, s_ref):
    zb = jnp.dot(h_ref[...], wt_ref[...], preferred_element_type=jnp.float32)
    zb = zb + b_ref[...]
    zb = jnp.where(jnp.abs(zb) == jnp.inf, jnp.float32(1e9), zb)
    z_ref[...] = zb
    sb = jnp.dot(zb, wa_ref[...], preferred_element_type=jnp.float32)
    s_ref[...] = sb + ba_ref[...]


def _project(hp, wt, bfc, wa, ba):
    return pl.pallas_call(
        _project_body,
        grid=(NPAD // BLK1,),
        in_specs=[
            pl.BlockSpec((BLK1, DIM), lambda i: (i, 0)),
            pl.BlockSpec((DIM, DIM), lambda i: (0, 0)),
            pl.BlockSpec((1, DIM), lambda i: (0, 0)),
            pl.BlockSpec((DIM, DIM), lambda i: (0, 0)),
            pl.BlockSpec((1, DIM), lambda i: (0, 0)),
        ],
        out_specs=[
            pl.BlockSpec((BLK1, DIM), lambda i: (i, 0)),
            pl.BlockSpec((BLK1, DIM), lambda i: (i, 0)),
        ],
        out_shape=[
            jax.ShapeDtypeStruct((NPAD, DIM), jnp.float32),
            jax.ShapeDtypeStruct((NPAD, DIM), jnp.float32),
        ],
    )(hp, wt, bfc, wa, ba)


@functools.partial(
    pl.kernel,
    out_type=(
        jax.ShapeDtypeStruct((EROWS, 128), jnp.float32),
        jax.ShapeDtypeStruct((NC, NPAD), jnp.float32),
        jax.ShapeDtypeStruct((NC, NPAD), jnp.float32),
    ),
    mesh=_mesh,
    scratch_types=(
        pltpu.VMEM((NPAD,), jnp.float32),
        pltpu.VMEM((NPAD,), jnp.float32),
        pltpu.VMEM((A_CHUNK, 128), jnp.int32),
        pltpu.VMEM((A_CHUNK, 128), jnp.int32),
        pltpu.VMEM((A_CHUNK, 128), jnp.float32),
        pltpu.VMEM((A_CHUNK, 128), jnp.float32),
        pltpu.VMEM((A_CHUNK, 128), jnp.float32),
        pltpu.VMEM((640,), jnp.float32),
        pltpu.VMEM_SHARED((NPAD,), jnp.float32),
        pltpu.VMEM_SHARED((NPAD,), jnp.float32),
    ),
)
def _sc_edge_logits(s1_hbm, s2_hbm, src_hbm, dst_hbm, sig_hbm,
                    coef_hbm, esum_hbm, bsum_hbm,
                    s1_t, s2_t, src_b, dst_b, sig_b, p_b, coef_b,
                    zb1, esum_sh, bsum_sh):
    cid = lax.axis_index("c")
    sid = lax.axis_index("s")
    wid = cid * NS + sid

    def zloop(i, carry):
        zb1[pl.ds(i * LANES, LANES)] = jnp.zeros((LANES,), jnp.float32)
        return carry
    lax.fori_loop(0, 640 // LANES, zloop, 0)
    base = sid * NSLICE
    pltpu.sync_copy(zb1.at[pl.ds(0, NSLICE)], esum_sh.at[pl.ds(base, NSLICE)])
    pltpu.sync_copy(zb1.at[pl.ds(0, NSLICE)], bsum_sh.at[pl.ds(base, NSLICE)])
    pltpu.sync_copy(s1_hbm, s1_t)
    pltpu.sync_copy(s2_hbm, s2_t)
    plsc.subcore_barrier()

    row0 = wid * ROWS_PER_W
    for ck in range(ROWS_PER_W // A_CHUNK):
        r0 = row0 + ck * A_CHUNK
        pltpu.sync_copy(src_hbm.at[pl.ds(r0, A_CHUNK)], src_b)
        pltpu.sync_copy(dst_hbm.at[pl.ds(r0, A_CHUNK)], dst_b)
        pltpu.sync_copy(sig_hbm.at[pl.ds(r0, A_CHUNK)], sig_b)

        def body(g, carry):
            r = g // (128 // LANES)
            c = (g % (128 // LANES)) * LANES
            si = src_b[r, pl.ds(c, LANES)]
            di = dst_b[r, pl.ds(c, LANES)]
            a = plsc.load_gather(s1_t, [si]) + plsc.load_gather(s2_t, [di])
            a = jnp.where(jnp.abs(a) == jnp.inf, jnp.float32(1e9), a)
            e = jnp.where(a > 0, a, a * jnp.float32(0.01))
            p = jnp.exp(e)
            p_b[r, pl.ds(c, LANES)] = p
            coef_b[r, pl.ds(c, LANES)] = p * sig_b[r, pl.ds(c, LANES)]
            return carry
        lax.fori_loop(0, A_CHUNK * (128 // LANES), body, 0)

        pltpu.sync_copy(coef_b, coef_hbm.at[pl.ds(r0, A_CHUNK)])
        for r in range(A_CHUNK):
            pltpu.sync_copy(p_b.at[r], esum_sh.at[dst_b.at[r]], add=True)
            pltpu.sync_copy(sig_b.at[r], bsum_sh.at[dst_b.at[r]], add=True)

    plsc.subcore_barrier()
    pltpu.sync_copy(esum_sh.at[pl.ds(base, NSLICE)],
                    esum_hbm.at[cid, pl.ds(base, NSLICE)])
    pltpu.sync_copy(bsum_sh.at[pl.ds(base, NSLICE)],
                    bsum_hbm.at[cid, pl.ds(base, NSLICE)])


@functools.partial(
    pl.kernel,
    out_type=jax.ShapeDtypeStruct((NC, NPAD, DIM), jnp.float32),
    mesh=_mesh,
    scratch_types=(
        pltpu.VMEM((B_CHUNK, 128), jnp.int32),
        pltpu.VMEM((B_CHUNK, 128), jnp.int32),
        pltpu.VMEM((B_CHUNK, 128), jnp.float32),
        pltpu.VMEM((B_CHUNK * 128, DIM), jnp.float32),
        pltpu.VMEM((64, DIM), jnp.float32),
        pltpu.VMEM_SHARED((NPAD, DIM), jnp.float32),
        pltpu.SemaphoreType.DMA,
    ),
)
def _sc_aggregate(z_hbm, src_hbm, dst_hbm, coef_hbm, hpart_hbm,
                  src_b, dst_b, coef_b, zr, zb, hacc_sh, sem):
    cid = lax.axis_index("c")
    sid = lax.axis_index("s")
    wid = cid * NS + sid

    def zloop(i, carry):
        for l in range(DIM // LANES):
            zb[i, pl.ds(l * LANES, LANES)] = jnp.zeros((LANES,), jnp.float32)
        return carry
    lax.fori_loop(0, 64, zloop, 0)
    base = sid * NSLICE
    for q in range(NSLICE // 64):
        pltpu.sync_copy(zb, hacc_sh.at[pl.ds(base + q * 64, 64)])
    rem = NSLICE - (NSLICE // 64) * 64
    if rem:
        pltpu.sync_copy(zb.at[pl.ds(0, rem)],
                        hacc_sh.at[pl.ds(base + NSLICE - rem, rem)])
    plsc.subcore_barrier()

    row0 = wid * ROWS_PER_W
    for ck in range(ROWS_PER_W // B_CHUNK):
        r0 = row0 + ck * B_CHUNK
        pltpu.sync_copy(src_hbm.at[pl.ds(r0, B_CHUNK)], src_b)
        pltpu.sync_copy(dst_hbm.at[pl.ds(r0, B_CHUNK)], dst_b)
        pltpu.sync_copy(coef_hbm.at[pl.ds(r0, B_CHUNK)], coef_b)
        for r in range(B_CHUNK):
            pltpu.async_copy(z_hbm.at[src_b.at[r]],
                             zr.at[pl.ds(r * 128, 128)], sem).wait()

        def scale(j, carry):
            r = j // 128
            cval = coef_b[r, j % 128]
            for l in range(DIM // LANES):
                zr[j, pl.ds(l * LANES, LANES)] = (
                    zr[j, pl.ds(l * LANES, LANES)] * cval)
            return carry
        lax.fori_loop(0, B_CHUNK * 128, scale, 0)

        for r in range(B_CHUNK):
            pltpu.sync_copy(zr.at[pl.ds(r * 128, 128)],
                            hacc_sh.at[dst_b.at[r]], add=True)

    plsc.subcore_barrier()
    pltpu.sync_copy(hacc_sh.at[pl.ds(base, NSLICE)],
                    hpart_hbm.at[cid, pl.ds(base, NSLICE)])


BLK3 = 400

def _combine_body(hp_ref, es_ref, bs_ref, o_ref):
    hp = hp_ref[0] + hp_ref[1]
    es = es_ref[0] + es_ref[1]
    bs = bs_ref[0] + bs_ref[1]
    den = es * (bs + jnp.float32(1e-6))
    t = jnp.where(es > 0, hp / den, jnp.float32(0.0))
    t = jnp.where(jnp.abs(t) == jnp.inf, jnp.float32(1e9), t)
    o_ref[...] = jnp.where(t > 0, t, jnp.exp(t) - jnp.float32(1.0))


def _combine(hpart, esum3, bsum3):
    return pl.pallas_call(
        _combine_body,
        grid=(N // BLK3,),
        in_specs=[
            pl.BlockSpec((NC, BLK3, DIM), lambda i: (0, i, 0)),
            pl.BlockSpec((NC, BLK3, 1), lambda i: (0, i, 0)),
            pl.BlockSpec((NC, BLK3, 1), lambda i: (0, i, 0)),
        ],
        out_specs=pl.BlockSpec((BLK3, DIM), lambda i: (i, 0)),
        out_shape=jax.ShapeDtypeStruct((N, DIM), jnp.float32),
    )(hpart, esum3, bsum3)


def kernel(h, edge_index, sigma_GD, W_fc, b_fc, W_attn, b_attn):
    f32 = jnp.float32
    src = edge_index[0].astype(jnp.int32)
    dst = edge_index[1].astype(jnp.int32)
    sig = sigma_GD.reshape(-1).astype(f32)
    pad_e = EPAD - E
    srcp = jnp.concatenate(
        [src, jnp.full((pad_e,), N, jnp.int32)]).reshape(EROWS, 128)
    dstp = jnp.concatenate(
        [dst, jnp.full((pad_e,), N, jnp.int32)]).reshape(EROWS, 128)
    sigp = jnp.concatenate(
        [sig, jnp.zeros((pad_e,), f32)]).reshape(EROWS, 128)
    hp = jnp.pad(h.astype(f32), ((0, NPAD - N), (0, 0)))
    wt = W_fc.T.astype(f32)
    bfc = b_fc.reshape(1, DIM).astype(f32)
    w12 = jnp.stack([W_attn[0, :DIM], W_attn[0, DIM:]], axis=1)
    wa = jnp.pad(w12, ((0, 0), (0, DIM - 2))).astype(f32)
    ba = jnp.zeros((1, DIM), f32).at[0, 1].set(b_attn[0])

    z, s = _project(hp, wt, bfc, wa, ba)
    s1 = s[:, 0]
    s2 = s[:, 1]
    coef, esum_p, bsum_p = _sc_edge_logits(s1, s2, srcp, dstp, sigp)
    hpart = _sc_aggregate(z, srcp, dstp, coef)
    return _combine(hpart,
                    esum_p.reshape(NC, NPAD, 1),
                    bsum_p.reshape(NC, NPAD, 1))


# speedup vs baseline: 13.5875x; 13.5875x over previous
import functools

import jax
import jax.numpy as jnp
from jax import lax
from jax.experimental import pallas as pl
from jax.experimental.pallas import tpu as pltpu
from jax.experimental.pallas import tpu_sc as plsc

N = 10000
E = 320000
DIM = 128
NC = 2
NS = 16
NW = NC * NS
LANES = 16

NPAD = 10112
NSLICE = NPAD // NS
ROWS_PER_W = 80
EROWS = NW * ROWS_PER_W
EPAD = EROWS * 128

A_CHUNK = 8
B_CHUNK = 2

_mesh = plsc.VectorSubcoreMesh(
    core_axis_name="c", subcore_axis_name="s", num_cores=NC, num_subcores=NS)


BLK1 = 632

def _project_body(h_ref, wt_ref, b_ref, wa_ref, ba_ref, z_ref, s_ref):
    zb = jnp.dot(h_ref[...], wt_ref[...], preferred_element_type=jnp.float32)
    zb = zb + b_ref[...]
    zb = jnp.where(jnp.abs(zb) == jnp.inf, jnp.float32(1e9), zb)
    z_ref[...] = zb
    sb = jnp.dot(zb, wa_ref[...], preferred_element_type=jnp.float32)
    s_ref[...] = sb + ba_ref[...]


def _project(hp, wt, bfc, wa, ba):
    return pl.pallas_call(
        _project_body,
        grid=(NPAD // BLK1,),
        in_specs=[
            pl.BlockSpec((BLK1, DIM), lambda i: (i, 0)),
            pl.BlockSpec((DIM, DIM), lambda i: (0, 0)),
            pl.BlockSpec((1, DIM), lambda i: (0, 0)),
            pl.BlockSpec((DIM, DIM), lambda i: (0, 0)),
            pl.BlockSpec((1, DIM), lambda i: (0, 0)),
        ],
        out_specs=[
            pl.BlockSpec((BLK1, DIM), lambda i: (i, 0)),
            pl.BlockSpec((BLK1, DIM), lambda i: (i, 0)),
        ],
        out_shape=[
            jax.ShapeDtypeStruct((NPAD, DIM), jnp.float32),
            jax.ShapeDtypeStruct((NPAD, DIM), jnp.float32),
        ],
    )(hp, wt, bfc, wa, ba)


@functools.partial(
    pl.kernel,
    out_type=(
        jax.ShapeDtypeStruct((EROWS, 128), jnp.float32),
        jax.ShapeDtypeStruct((NC * NPAD,), jnp.float32),
        jax.ShapeDtypeStruct((NC * NPAD,), jnp.float32),
    ),
    mesh=_mesh,
    scratch_types=(
        pltpu.VMEM((NPAD,), jnp.float32),
        pltpu.VMEM((NPAD,), jnp.float32),
        pltpu.VMEM((A_CHUNK, 128), jnp.int32),
        pltpu.VMEM((A_CHUNK, 128), jnp.int32),
        pltpu.VMEM((A_CHUNK, 128), jnp.float32),
        pltpu.VMEM((A_CHUNK, 128), jnp.float32),
        pltpu.VMEM((A_CHUNK, 128), jnp.float32),
        pltpu.VMEM((640,), jnp.float32),
        pltpu.VMEM_SHARED((NPAD,), jnp.float32),
        pltpu.VMEM_SHARED((NPAD,), jnp.float32),
    ),
    compiler_params=pltpu.CompilerParams(needs_layout_passes=False),
)
def _sc_edge_logits(s1_hbm, s2_hbm, src_hbm, dst_hbm, sig_hbm,
                    coef_hbm, esum_hbm, bsum_hbm,
                    s1_t, s2_t, src_b, dst_b, sig_b, p_b, coef_b,
                    zb1, esum_sh, bsum_sh):
    cid = lax.axis_index("c")
    sid = lax.axis_index("s")
    wid = cid * NS + sid

    def zloop(i, carry):
        zb1[pl.ds(i * LANES, LANES)] = jnp.zeros((LANES,), jnp.float32)
        return carry
    lax.fori_loop(0, 640 // LANES, zloop, 0)
    base = sid * NSLICE
    pltpu.sync_copy(zb1.at[pl.ds(0, NSLICE)], esum_sh.at[pl.ds(base, NSLICE)])
    pltpu.sync_copy(zb1.at[pl.ds(0, NSLICE)], bsum_sh.at[pl.ds(base, NSLICE)])
    pltpu.sync_copy(s1_hbm, s1_t)
    pltpu.sync_copy(s2_hbm, s2_t)
    plsc.subcore_barrier()

    row0 = wid * ROWS_PER_W
    for ck in range(ROWS_PER_W // A_CHUNK):
        r0 = row0 + ck * A_CHUNK
        pltpu.sync_copy(src_hbm.at[pl.ds(r0, A_CHUNK)], src_b)
        pltpu.sync_copy(dst_hbm.at[pl.ds(r0, A_CHUNK)], dst_b)
        pltpu.sync_copy(sig_hbm.at[pl.ds(r0, A_CHUNK)], sig_b)

        def body(g, carry):
            r = g // (128 // LANES)
            c = (g % (128 // LANES)) * LANES
            si = src_b[r, pl.ds(c, LANES)]
            di = dst_b[r, pl.ds(c, LANES)]
            a = plsc.load_gather(s1_t, [si]) + plsc.load_gather(s2_t, [di])
            a = jnp.where(jnp.abs(a) == jnp.inf, jnp.float32(1e9), a)
            e = jnp.where(a > 0, a, a * jnp.float32(0.01))
            p = jnp.exp(e)
            p_b[r, pl.ds(c, LANES)] = p
            coef_b[r, pl.ds(c, LANES)] = p * sig_b[r, pl.ds(c, LANES)]
            return carry
        lax.fori_loop(0, A_CHUNK * (128 // LANES), body, 0)

        pltpu.sync_copy(coef_b, coef_hbm.at[pl.ds(r0, A_CHUNK)])
        for r in range(A_CHUNK):
            pltpu.sync_copy(p_b.at[r], esum_sh.at[dst_b.at[r]], add=True)
            pltpu.sync_copy(sig_b.at[r], bsum_sh.at[dst_b.at[r]], add=True)

    plsc.subcore_barrier()
    pltpu.sync_copy(esum_sh.at[pl.ds(base, NSLICE)], zb1.at[pl.ds(0, NSLICE)])
    pltpu.sync_copy(zb1.at[pl.ds(0, NSLICE)],
                    esum_hbm.at[pl.ds(cid * NPAD + base, NSLICE)])
    pltpu.sync_copy(bsum_sh.at[pl.ds(base, NSLICE)], zb1.at[pl.ds(0, NSLICE)])
    pltpu.sync_copy(zb1.at[pl.ds(0, NSLICE)],
                    bsum_hbm.at[pl.ds(cid * NPAD + base, NSLICE)])


@functools.partial(
    pl.kernel,
    out_type=jax.ShapeDtypeStruct((NC, NPAD, DIM), jnp.float32),
    mesh=_mesh,
    scratch_types=(
        pltpu.VMEM((B_CHUNK, 128), jnp.int32),
        pltpu.VMEM((B_CHUNK, 128), jnp.int32),
        pltpu.VMEM((B_CHUNK, 128), jnp.float32),
        pltpu.VMEM((B_CHUNK * 128, DIM), jnp.float32),
        pltpu.VMEM((64, DIM), jnp.float32),
        pltpu.VMEM_SHARED((NPAD, DIM), jnp.float32),
        pltpu.SemaphoreType.DMA,
    ),
    compiler_params=pltpu.CompilerParams(needs_layout_passes=False),
)
def _sc_aggregate(z_hbm, src_hbm, dst_hbm, coef_hbm, hpart_hbm,
                  src_b, dst_b, coef_b, zr, zb, hacc_sh, sem):
    cid = lax.axis_index("c")
    sid = lax.axis_index("s")
    wid = cid * NS + sid

    def zloop(i, carry):
        for l in range(DIM // LANES):
            zb[i, pl.ds(l * LANES, LANES)] = jnp.zeros((LANES,), jnp.float32)
        return carry
    lax.fori_loop(0, 64, zloop, 0)
    base = sid * NSLICE
    for q in range(NSLICE // 64):
        pltpu.sync_copy(zb, hacc_sh.at[pl.ds(base + q * 64, 64)])
    rem = NSLICE - (NSLICE // 64) * 64
    if rem:
        pltpu.sync_copy(zb.at[pl.ds(0, rem)],
                        hacc_sh.at[pl.ds(base + NSLICE - rem, rem)])
    plsc.subcore_barrier()

    row0 = wid * ROWS_PER_W
    for ck in range(ROWS_PER_W // B_CHUNK):
        r0 = row0 + ck * B_CHUNK
        pltpu.sync_copy(src_hbm.at[pl.ds(r0, B_CHUNK)], src_b)
        pltpu.sync_copy(dst_hbm.at[pl.ds(r0, B_CHUNK)], dst_b)
        pltpu.sync_copy(coef_hbm.at[pl.ds(r0, B_CHUNK)], coef_b)
        for r in range(B_CHUNK):
            pltpu.async_copy(z_hbm.at[src_b.at[r]],
                             zr.at[pl.ds(r * 128, 128)], sem).wait()

        def scale(j, carry):
            r = j // 128
            jc = j % 128
            cval = plsc.load_gather(
                coef_b, [jnp.full((LANES,), r, jnp.int32),
                         jnp.full((LANES,), jc, jnp.int32)])
            for l in range(DIM // LANES):
                zr[j, pl.ds(l * LANES, LANES)] = (
                    zr[j, pl.ds(l * LANES, LANES)] * cval)
            return carry
        lax.fori_loop(0, B_CHUNK * 128, scale, 0)

        for r in range(B_CHUNK):
            pltpu.sync_copy(zr.at[pl.ds(r * 128, 128)],
                            hacc_sh.at[dst_b.at[r]], add=True)

    plsc.subcore_barrier()
    out_rows = B_CHUNK * 128
    off = 0
    while off < NSLICE:
        n = min(out_rows, NSLICE - off)
        pltpu.sync_copy(hacc_sh.at[pl.ds(base + off, n)], zr.at[pl.ds(0, n)])
        pltpu.sync_copy(zr.at[pl.ds(0, n)],
                        hpart_hbm.at[cid, pl.ds(base + off, n)])
        off += n


BLK3 = 400

def _combine_body(hp_ref, es_ref, bs_ref, o_ref):
    hp = hp_ref[0] + hp_ref[1]
    es = es_ref[0] + es_ref[1]
    bs = bs_ref[0] + bs_ref[1]
    den = es * (bs + jnp.float32(1e-6))
    t = jnp.where(es > 0, hp / den, jnp.float32(0.0))
    t = jnp.where(jnp.abs(t) == jnp.inf, jnp.float32(1e9), t)
    o_ref[...] = jnp.where(t > 0, t, jnp.exp(t) - jnp.float32(1.0))


def _combine(hpart, esum3, bsum3):
    return pl.pallas_call(
        _combine_body,
        grid=(N // BLK3,),
        in_specs=[
            pl.BlockSpec((NC, BLK3, DIM), lambda i: (0, i, 0)),
            pl.BlockSpec((NC, BLK3, 1), lambda i: (0, i, 0)),
            pl.BlockSpec((NC, BLK3, 1), lambda i: (0, i, 0)),
        ],
        out_specs=pl.BlockSpec((BLK3, DIM), lambda i: (i, 0)),
        out_shape=jax.ShapeDtypeStruct((N, DIM), jnp.float32),
    )(hpart, esum3, bsum3)


def kernel(h, edge_index, sigma_GD, W_fc, b_fc, W_attn, b_attn):
    f32 = jnp.float32
    src = edge_index[0].astype(jnp.int32)
    dst = edge_index[1].astype(jnp.int32)
    sig = sigma_GD.reshape(-1).astype(f32)
    pad_e = EPAD - E
    srcp = jnp.concatenate(
        [src, jnp.full((pad_e,), N, jnp.int32)]).reshape(EROWS, 128)
    dstp = jnp.concatenate(
        [dst, jnp.full((pad_e,), N, jnp.int32)]).reshape(EROWS, 128)
    sigp = jnp.concatenate(
        [sig, jnp.zeros((pad_e,), f32)]).reshape(EROWS, 128)
    hp = jnp.pad(h.astype(f32), ((0, NPAD - N), (0, 0)))
    wt = W_fc.T.astype(f32)
    bfc = b_fc.reshape(1, DIM).astype(f32)
    w12 = jnp.stack([W_attn[0, :DIM], W_attn[0, DIM:]], axis=1)
    wa = jnp.pad(w12, ((0, 0), (0, DIM - 2))).astype(f32)
    ba = jnp.zeros((1, DIM), f32).at[0, 1].set(b_attn[0])

    z, s = _project(hp, wt, bfc, wa, ba)
    s1 = s[:, 0]
    s2 = s[:, 1]
    coef, esum_p, bsum_p = _sc_edge_logits(s1, s2, srcp, dstp, sigp)
    hpart = _sc_aggregate(z, srcp, dstp, coef)
    return _combine(hpart,
                    esum_p.reshape(NC, NPAD, 1),
                    bsum_p.reshape(NC, NPAD, 1))


# speedup vs baseline: 15.3765x; 1.1317x over previous
import functools

import jax
import jax.numpy as jnp
from jax import lax
from jax.experimental import pallas as pl
from jax.experimental.pallas import tpu as pltpu
from jax.experimental.pallas import tpu_sc as plsc

N = 10000
E = 320000
DIM = 128
NC = 2
NS = 16
NW = NC * NS
LANES = 16

NPAD = 10112
NSLICE = NPAD // NS
ROWS_PER_W = 80
EROWS = NW * ROWS_PER_W
EPAD = EROWS * 128

A_CHUNK = 8
B_CHUNK = 2

_mesh = plsc.VectorSubcoreMesh(
    core_axis_name="c", subcore_axis_name="s", num_cores=NC, num_subcores=NS)


BLK1 = 632

def _project_body(h_ref, wt_ref, b_ref, wa_ref, ba_ref, z_ref, s_ref):
    zb = jnp.dot(h_ref[...], wt_ref[...], preferred_element_type=jnp.float32)
    zb = zb + b_ref[...]
    zb = jnp.where(jnp.abs(zb) == jnp.inf, jnp.float32(1e9), zb)
    z_ref[...] = zb
    sb = jnp.dot(zb, wa_ref[...], preferred_element_type=jnp.float32)
    s_ref[...] = sb + ba_ref[...]


def _project(hp, wt, bfc, wa, ba):
    return pl.pallas_call(
        _project_body,
        grid=(NPAD // BLK1,),
        in_specs=[
            pl.BlockSpec((BLK1, DIM), lambda i: (i, 0)),
            pl.BlockSpec((DIM, DIM), lambda i: (0, 0)),
            pl.BlockSpec((1, DIM), lambda i: (0, 0)),
            pl.BlockSpec((DIM, DIM), lambda i: (0, 0)),
            pl.BlockSpec((1, DIM), lambda i: (0, 0)),
        ],
        out_specs=[
            pl.BlockSpec((BLK1, DIM), lambda i: (i, 0)),
            pl.BlockSpec((BLK1, DIM), lambda i: (i, 0)),
        ],
        out_shape=[
            jax.ShapeDtypeStruct((NPAD, DIM), jnp.float32),
            jax.ShapeDtypeStruct((NPAD, DIM), jnp.float32),
        ],
    )(hp, wt, bfc, wa, ba)


@functools.partial(
    pl.kernel,
    out_type=(
        jax.ShapeDtypeStruct((EROWS, 128), jnp.float32),
        jax.ShapeDtypeStruct((NC * NPAD,), jnp.float32),
        jax.ShapeDtypeStruct((NC * NPAD,), jnp.float32),
    ),
    mesh=_mesh,
    scratch_types=(
        pltpu.VMEM((NPAD,), jnp.float32),
        pltpu.VMEM((NPAD,), jnp.float32),
        pltpu.VMEM((A_CHUNK, 128), jnp.int32),
        pltpu.VMEM((A_CHUNK, 128), jnp.int32),
        pltpu.VMEM((A_CHUNK, 128), jnp.float32),
        pltpu.VMEM((A_CHUNK, 128), jnp.float32),
        pltpu.VMEM((A_CHUNK, 128), jnp.float32),
        pltpu.VMEM((640,), jnp.float32),
        pltpu.VMEM_SHARED((NPAD,), jnp.float32),
        pltpu.VMEM_SHARED((NPAD,), jnp.float32),
    ),
    compiler_params=pltpu.CompilerParams(needs_layout_passes=False),
)
def _sc_edge_logits(s1_hbm, s2_hbm, src_hbm, dst_hbm, sig_hbm,
                    coef_hbm, esum_hbm, bsum_hbm,
                    s1_t, s2_t, src_b, dst_b, sig_b, p_b, coef_b,
                    zb1, esum_sh, bsum_sh):
    cid = lax.axis_index("c")
    sid = lax.axis_index("s")
    wid = cid * NS + sid

    def zloop(i, carry):
        zb1[pl.ds(i * LANES, LANES)] = jnp.zeros((LANES,), jnp.float32)
        return carry
    lax.fori_loop(0, 640 // LANES, zloop, 0)
    base = sid * NSLICE
    pltpu.sync_copy(zb1.at[pl.ds(0, NSLICE)], esum_sh.at[pl.ds(base, NSLICE)])
    pltpu.sync_copy(zb1.at[pl.ds(0, NSLICE)], bsum_sh.at[pl.ds(base, NSLICE)])
    pltpu.sync_copy(s1_hbm, s1_t)
    pltpu.sync_copy(s2_hbm, s2_t)
    plsc.subcore_barrier()

    row0 = wid * ROWS_PER_W
    for ck in range(ROWS_PER_W // A_CHUNK):
        r0 = row0 + ck * A_CHUNK
        pltpu.sync_copy(src_hbm.at[pl.ds(r0, A_CHUNK)], src_b)
        pltpu.sync_copy(dst_hbm.at[pl.ds(r0, A_CHUNK)], dst_b)
        pltpu.sync_copy(sig_hbm.at[pl.ds(r0, A_CHUNK)], sig_b)

        def body(g, carry):
            r = g // (128 // LANES)
            c = (g % (128 // LANES)) * LANES
            si = src_b[r, pl.ds(c, LANES)]
            di = dst_b[r, pl.ds(c, LANES)]
            a = plsc.load_gather(s1_t, [si]) + plsc.load_gather(s2_t, [di])
            a = jnp.where(jnp.abs(a) == jnp.inf, jnp.float32(1e9), a)
            e = jnp.where(a > 0, a, a * jnp.float32(0.01))
            p = jnp.exp(e)
            p_b[r, pl.ds(c, LANES)] = p
            coef_b[r, pl.ds(c, LANES)] = p * sig_b[r, pl.ds(c, LANES)]
            return carry
        lax.fori_loop(0, A_CHUNK * (128 // LANES), body, 0)

        pltpu.sync_copy(coef_b, coef_hbm.at[pl.ds(r0, A_CHUNK)])
        for r in range(A_CHUNK):
            pltpu.sync_copy(p_b.at[r], esum_sh.at[dst_b.at[r]], add=True)
            pltpu.sync_copy(sig_b.at[r], bsum_sh.at[dst_b.at[r]], add=True)

    plsc.subcore_barrier()
    pltpu.sync_copy(esum_sh.at[pl.ds(base, NSLICE)], zb1.at[pl.ds(0, NSLICE)])
    pltpu.sync_copy(zb1.at[pl.ds(0, NSLICE)],
                    esum_hbm.at[pl.ds(cid * NPAD + base, NSLICE)])
    pltpu.sync_copy(bsum_sh.at[pl.ds(base, NSLICE)], zb1.at[pl.ds(0, NSLICE)])
    pltpu.sync_copy(zb1.at[pl.ds(0, NSLICE)],
                    bsum_hbm.at[pl.ds(cid * NPAD + base, NSLICE)])


N_CHUNKS = ROWS_PER_W


@functools.partial(
    pl.kernel,
    out_type=jax.ShapeDtypeStruct((NC, NPAD, DIM), jnp.float32),
    mesh=_mesh,
    scratch_types=(
        pltpu.VMEM((2, 2, 128), jnp.int32),
        pltpu.VMEM((ROWS_PER_W, 128), jnp.int32),
        pltpu.VMEM((128, DIM), jnp.float32),
        pltpu.VMEM((128, DIM), jnp.float32),
        pltpu.VMEM_SHARED((NPAD, DIM), jnp.float32),
        pltpu.SemaphoreType.DMA,
        pltpu.SemaphoreType.DMA,
        pltpu.SemaphoreType.DMA,
    ),
    compiler_params=pltpu.CompilerParams(needs_layout_passes=False),
)
def _sc_aggregate(z_hbm, edata_hbm, dst_hbm, hpart_hbm,
                  ebuf, dst_a, zr0, zr1, hacc_sh, esem, gsem, ssem):
    cid = lax.axis_index("c")
    sid = lax.axis_index("s")
    wid = cid * NS + sid
    zbufs = (zr0, zr1)
    row0 = wid * ROWS_PER_W

    dld = pltpu.async_copy(dst_hbm.at[pl.ds(row0, ROWS_PER_W)], dst_a, esem)
    el0 = pltpu.async_copy(edata_hbm.at[row0], ebuf.at[0], esem)

    def zloop(i, carry):
        for l in range(DIM // LANES):
            zr0[i, pl.ds(l * LANES, LANES)] = jnp.zeros((LANES,), jnp.float32)
        return carry
    lax.fori_loop(0, 128, zloop, 0)
    base = sid * NSLICE
    off = 0
    while off < NSLICE:
        n = min(128, NSLICE - off)
        pltpu.sync_copy(zr0.at[pl.ds(0, n)], hacc_sh.at[pl.ds(base + off, n)])
        off += n
    plsc.subcore_barrier()
    dld.wait()
    el0.wait()

    def issue_eload(ck):
        return pltpu.async_copy(edata_hbm.at[row0 + ck], ebuf.at[ck % 2], esem)

    def issue_gather(ck):
        return pltpu.async_copy(z_hbm.at[ebuf.at[ck % 2, 0]],
                                zbufs[ck % 2], gsem)

    def issue_scatter(ck):
        return pltpu.async_copy(zbufs[ck % 2], hacc_sh.at[dst_a.at[ck]],
                                ssem, add=True)

    def scale(ck):
        buf = zbufs[ck % 2]
        s = ck % 2

        def body(j, carry):
            cbits = plsc.load_gather(
                ebuf, [jnp.full((LANES,), s, jnp.int32),
                       jnp.full((LANES,), 1, jnp.int32),
                       jnp.full((LANES,), j, jnp.int32)])
            cval = plsc.bitcast(cbits, jnp.float32)
            for l in range(DIM // LANES):
                buf[j, pl.ds(l * LANES, LANES)] = (
                    buf[j, pl.ds(l * LANES, LANES)] * cval)
            return carry
        lax.fori_loop(0, 128, body, 0)

    gd = {0: issue_gather(0)}
    ed = {1: issue_eload(1)} if N_CHUNKS > 1 else {}
    sd = {}
    for ck in range(N_CHUNKS):
        gd.pop(ck).wait()
        if ck + 1 in ed:
            ed.pop(ck + 1).wait()
        scale(ck)
        if ck >= 1:
            sd.pop(ck - 1).wait()
        if ck + 1 < N_CHUNKS:
            gd[ck + 1] = issue_gather(ck + 1)
        sd[ck] = issue_scatter(ck)
        if ck + 2 < N_CHUNKS:
            ed[ck + 2] = issue_eload(ck + 2)
    sd.pop(N_CHUNKS - 1).wait()

    plsc.subcore_barrier()
    off = 0
    while off < NSLICE:
        n = min(128, NSLICE - off)
        pltpu.sync_copy(hacc_sh.at[pl.ds(base + off, n)], zr0.at[pl.ds(0, n)])
        pltpu.sync_copy(zr0.at[pl.ds(0, n)],
                        hpart_hbm.at[cid, pl.ds(base + off, n)])
        off += n


BLK3 = 400

def _combine_body(hp_ref, es_ref, bs_ref, o_ref):
    hp = hp_ref[0] + hp_ref[1]
    es = es_ref[0] + es_ref[1]
    bs = bs_ref[0] + bs_ref[1]
    den = es * (bs + jnp.float32(1e-6))
    t = jnp.where(es > 0, hp / den, jnp.float32(0.0))
    t = jnp.where(jnp.abs(t) == jnp.inf, jnp.float32(1e9), t)
    o_ref[...] = jnp.where(t > 0, t, jnp.exp(t) - jnp.float32(1.0))


def _combine(hpart, esum3, bsum3):
    return pl.pallas_call(
        _combine_body,
        grid=(N // BLK3,),
        in_specs=[
            pl.BlockSpec((NC, BLK3, DIM), lambda i: (0, i, 0)),
            pl.BlockSpec((NC, BLK3, 1), lambda i: (0, i, 0)),
            pl.BlockSpec((NC, BLK3, 1), lambda i: (0, i, 0)),
        ],
        out_specs=pl.BlockSpec((BLK3, DIM), lambda i: (i, 0)),
        out_shape=jax.ShapeDtypeStruct((N, DIM), jnp.float32),
    )(hpart, esum3, bsum3)


def kernel(h, edge_index, sigma_GD, W_fc, b_fc, W_attn, b_attn):
    f32 = jnp.float32
    src = edge_index[0].astype(jnp.int32)
    dst = edge_index[1].astype(jnp.int32)
    sig = sigma_GD.reshape(-1).astype(f32)
    pad_e = EPAD - E
    srcp = jnp.concatenate(
        [src, jnp.full((pad_e,), N, jnp.int32)]).reshape(EROWS, 128)
    dstp = jnp.concatenate(
        [dst, jnp.full((pad_e,), N, jnp.int32)]).reshape(EROWS, 128)
    sigp = jnp.concatenate(
        [sig, jnp.zeros((pad_e,), f32)]).reshape(EROWS, 128)
    hp = jnp.pad(h.astype(f32), ((0, NPAD - N), (0, 0)))
    wt = W_fc.T.astype(f32)
    bfc = b_fc.reshape(1, DIM).astype(f32)
    w12 = jnp.stack([W_attn[0, :DIM], W_attn[0, DIM:]], axis=1)
    wa = jnp.pad(w12, ((0, 0), (0, DIM - 2))).astype(f32)
    ba = jnp.zeros((1, DIM), f32).at[0, 1].set(b_attn[0])

    z, s = _project(hp, wt, bfc, wa, ba)
    s1 = s[:, 0]
    s2 = s[:, 1]
    coef, esum_p, bsum_p = _sc_edge_logits(s1, s2, srcp, dstp, sigp)
    edata = jnp.stack([srcp, lax.bitcast_convert_type(coef, jnp.int32)],
                      axis=1)
    hpart = _sc_aggregate(z, edata, dstp)
    return _combine(hpart,
                    esum_p.reshape(NC, NPAD, 1),
                    bsum_p.reshape(NC, NPAD, 1))


# speedup vs baseline: 30.9290x; 2.0115x over previous
import functools

import jax
import jax.numpy as jnp
from jax import lax
from jax.experimental import pallas as pl
from jax.experimental.pallas import tpu as pltpu
from jax.experimental.pallas import tpu_sc as plsc

N = 10000
E = 320000
DIM = 128
NC = 2
NS = 16
NW = NC * NS
LANES = 16

NPAD = 10112
NSLICE = NPAD // NS
ROWS_PER_W = 80
EROWS = NW * ROWS_PER_W
EPAD = EROWS * 128

A_CHUNK = 8

_mesh = plsc.VectorSubcoreMesh(
    core_axis_name="c", subcore_axis_name="s", num_cores=NC, num_subcores=NS)


BLK1 = 632

def _project_body(h_ref, wt_ref, b_ref, wa_ref, ba_ref, z_ref, s_ref):
    zb = jnp.dot(h_ref[...], wt_ref[...], preferred_element_type=jnp.float32)
    zb = zb + b_ref[...]
    zb = jnp.where(jnp.abs(zb) == jnp.inf, jnp.float32(1e9), zb)
    z_ref[...] = zb
    sb = jnp.dot(zb, wa_ref[...], preferred_element_type=jnp.float32)
    s_ref[...] = sb + ba_ref[...]


def _project(hp, wt, bfc, wa, ba):
    return pl.pallas_call(
        _project_body,
        grid=(NPAD // BLK1,),
        in_specs=[
            pl.BlockSpec((BLK1, DIM), lambda i: (i, 0)),
            pl.BlockSpec((DIM, DIM), lambda i: (0, 0)),
            pl.BlockSpec((1, DIM), lambda i: (0, 0)),
            pl.BlockSpec((DIM, DIM), lambda i: (0, 0)),
            pl.BlockSpec((1, DIM), lambda i: (0, 0)),
        ],
        out_specs=[
            pl.BlockSpec((BLK1, DIM), lambda i: (i, 0)),
            pl.BlockSpec((BLK1, DIM), lambda i: (i, 0)),
        ],
        out_shape=[
            jax.ShapeDtypeStruct((NPAD, DIM), jnp.float32),
            jax.ShapeDtypeStruct((NPAD, DIM), jnp.float32),
        ],
    )(hp, wt, bfc, wa, ba)


@functools.partial(
    pl.kernel,
    out_type=(
        jax.ShapeDtypeStruct((EROWS, 128), jnp.float32),
        jax.ShapeDtypeStruct((NC * NPAD,), jnp.float32),
        jax.ShapeDtypeStruct((NC * NPAD,), jnp.float32),
    ),
    mesh=_mesh,
    scratch_types=(
        pltpu.VMEM((NPAD,), jnp.float32),
        pltpu.VMEM((NPAD,), jnp.float32),
        pltpu.VMEM((A_CHUNK, 128), jnp.int32),
        pltpu.VMEM((A_CHUNK, 128), jnp.int32),
        pltpu.VMEM((A_CHUNK, 128), jnp.float32),
        pltpu.VMEM((A_CHUNK, 128), jnp.float32),
        pltpu.VMEM((A_CHUNK, 128), jnp.float32),
        pltpu.VMEM((640,), jnp.float32),
        pltpu.VMEM_SHARED((NPAD,), jnp.float32),
        pltpu.VMEM_SHARED((NPAD,), jnp.float32),
    ),
    compiler_params=pltpu.CompilerParams(needs_layout_passes=False),
)
def _sc_edge_logits(s1_hbm, s2_hbm, src_hbm, dst_hbm, sig_hbm,
                    coef_hbm, esum_hbm, bsum_hbm,
                    s1_t, s2_t, src_b, dst_b, sig_b, p_b, coef_b,
                    zb1, esum_sh, bsum_sh):
    cid = lax.axis_index("c")
    sid = lax.axis_index("s")
    wid = cid * NS + sid

    def zloop(i, carry):
        zb1[pl.ds(i * LANES, LANES)] = jnp.zeros((LANES,), jnp.float32)
        return carry
    lax.fori_loop(0, 640 // LANES, zloop, 0)
    base = sid * NSLICE
    pltpu.sync_copy(zb1.at[pl.ds(0, NSLICE)], esum_sh.at[pl.ds(base, NSLICE)])
    pltpu.sync_copy(zb1.at[pl.ds(0, NSLICE)], bsum_sh.at[pl.ds(base, NSLICE)])
    pltpu.sync_copy(s1_hbm, s1_t)
    pltpu.sync_copy(s2_hbm, s2_t)
    plsc.subcore_barrier()

    row0 = wid * ROWS_PER_W
    for ck in range(ROWS_PER_W // A_CHUNK):
        r0 = row0 + ck * A_CHUNK
        pltpu.sync_copy(src_hbm.at[pl.ds(r0, A_CHUNK)], src_b)
        pltpu.sync_copy(dst_hbm.at[pl.ds(r0, A_CHUNK)], dst_b)
        pltpu.sync_copy(sig_hbm.at[pl.ds(r0, A_CHUNK)], sig_b)

        def body(g, carry):
            r = g // (128 // LANES)
            c = (g % (128 // LANES)) * LANES
            si = src_b[r, pl.ds(c, LANES)]
            di = dst_b[r, pl.ds(c, LANES)]
            a = plsc.load_gather(s1_t, [si]) + plsc.load_gather(s2_t, [di])
            a = jnp.where(jnp.abs(a) == jnp.inf, jnp.float32(1e9), a)
            e = jnp.where(a > 0, a, a * jnp.float32(0.01))
            p = jnp.exp(e)
            p_b[r, pl.ds(c, LANES)] = p
            coef_b[r, pl.ds(c, LANES)] = p * sig_b[r, pl.ds(c, LANES)]
            return carry
        lax.fori_loop(0, A_CHUNK * (128 // LANES), body, 0)

        pltpu.sync_copy(coef_b, coef_hbm.at[pl.ds(r0, A_CHUNK)])
        for r in range(A_CHUNK):
            pltpu.sync_copy(p_b.at[r], esum_sh.at[dst_b.at[r]], add=True)
            pltpu.sync_copy(sig_b.at[r], bsum_sh.at[dst_b.at[r]], add=True)

    plsc.subcore_barrier()
    pltpu.sync_copy(esum_sh.at[pl.ds(base, NSLICE)], zb1.at[pl.ds(0, NSLICE)])
    pltpu.sync_copy(zb1.at[pl.ds(0, NSLICE)],
                    esum_hbm.at[pl.ds(cid * NPAD + base, NSLICE)])
    pltpu.sync_copy(bsum_sh.at[pl.ds(base, NSLICE)], zb1.at[pl.ds(0, NSLICE)])
    pltpu.sync_copy(zb1.at[pl.ds(0, NSLICE)],
                    bsum_hbm.at[pl.ds(cid * NPAD + base, NSLICE)])


N_CHUNKS = ROWS_PER_W


@functools.partial(
    pl.kernel,
    out_type=jax.ShapeDtypeStruct((NC, NPAD, DIM), jnp.float32),
    mesh=_mesh,
    scratch_types=(
        pltpu.VMEM((2, 2, 128), jnp.int32),
        pltpu.VMEM((ROWS_PER_W, 128), jnp.int32),
        pltpu.VMEM((128, DIM), jnp.float32),
        pltpu.VMEM((128, DIM), jnp.float32),
        pltpu.VMEM_SHARED((NPAD, DIM), jnp.float32),
        pltpu.SemaphoreType.DMA,
        pltpu.SemaphoreType.DMA,
        pltpu.SemaphoreType.DMA,
    ),
    compiler_params=pltpu.CompilerParams(needs_layout_passes=False),
)
def _sc_aggregate(z_hbm, edata_hbm, dst_hbm, hpart_hbm,
                  ebuf, dst_a, zr0, zr1, hacc_sh, esem, gsem, ssem):
    cid = lax.axis_index("c")
    sid = lax.axis_index("s")
    wid = cid * NS + sid
    zbufs = (zr0, zr1)
    row0 = wid * ROWS_PER_W

    def issue_eload(ck):
        return pltpu.async_copy(edata_hbm.at[row0 + ck], ebuf.at[ck % 2], esem)

    dld = pltpu.async_copy(dst_hbm.at[pl.ds(row0, ROWS_PER_W)], dst_a, esem)
    el = {ck: issue_eload(ck) for ck in range(2)}

    def zloop(i, carry):
        for l in range(DIM // LANES):
            zr0[i, pl.ds(l * LANES, LANES)] = jnp.zeros((LANES,), jnp.float32)
        return carry
    lax.fori_loop(0, 128, zloop, 0)
    base = sid * NSLICE
    off = 0
    while off < NSLICE:
        n = min(128, NSLICE - off)
        pltpu.sync_copy(zr0.at[pl.ds(0, n)], hacc_sh.at[pl.ds(base + off, n)])
        off += n
    plsc.subcore_barrier()
    dld.wait()

    def issue_gather(ck):
        return pltpu.async_copy(z_hbm.at[ebuf.at[ck % 2, 0]],
                                zbufs[ck % 2], gsem)

    def issue_scatter(ck):
        return pltpu.async_copy(zbufs[ck % 2], hacc_sh.at[dst_a.at[ck]],
                                ssem, add=True)

    def scale(ck):
        buf = zbufs[ck % 2]
        s = ck % 2

        def body(j, carry):
            cbits = plsc.load_gather(
                ebuf, [jnp.full((LANES,), s, jnp.int32),
                       jnp.full((LANES,), 1, jnp.int32),
                       jnp.full((LANES,), j, jnp.int32)])
            cval = plsc.bitcast(cbits, jnp.float32)
            for l in range(DIM // LANES):
                buf[j, pl.ds(l * LANES, LANES)] = (
                    buf[j, pl.ds(l * LANES, LANES)] * cval)
            return carry
        lax.fori_loop(0, 128, body, 0)

    el.pop(0).wait()
    gd = {0: issue_gather(0)}
    sd = {}
    for ck in range(N_CHUNKS):
        if ck >= 1:
            sd.pop(ck - 1).wait()
        if ck + 1 < N_CHUNKS:
            el.pop(ck + 1).wait()
            gd[ck + 1] = issue_gather(ck + 1)
        gd.pop(ck).wait()
        scale(ck)
        sd[ck] = issue_scatter(ck)
        if ck + 2 < N_CHUNKS:
            el[ck + 2] = issue_eload(ck + 2)
    sd.pop(N_CHUNKS - 1).wait()

    plsc.subcore_barrier()
    off = 0
    while off < NSLICE:
        n = min(128, NSLICE - off)
        pltpu.sync_copy(hacc_sh.at[pl.ds(base + off, n)], zr0.at[pl.ds(0, n)])
        pltpu.sync_copy(zr0.at[pl.ds(0, n)],
                        hpart_hbm.at[cid, pl.ds(base + off, n)])
        off += n


BLK3 = 400

def _combine_body(hp_ref, es_ref, bs_ref, o_ref):
    hp = hp_ref[0] + hp_ref[1]
    es = es_ref[0] + es_ref[1]
    bs = bs_ref[0] + bs_ref[1]
    den = es * (bs + jnp.float32(1e-6))
    t = jnp.where(es > 0, hp / den, jnp.float32(0.0))
    t = jnp.where(jnp.abs(t) == jnp.inf, jnp.float32(1e9), t)
    o_ref[...] = jnp.where(t > 0, t, jnp.exp(t) - jnp.float32(1.0))


def _combine(hpart, esum3, bsum3):
    return pl.pallas_call(
        _combine_body,
        grid=(N // BLK3,),
        in_specs=[
            pl.BlockSpec((NC, BLK3, DIM), lambda i: (0, i, 0)),
            pl.BlockSpec((NC, BLK3, 1), lambda i: (0, i, 0)),
            pl.BlockSpec((NC, BLK3, 1), lambda i: (0, i, 0)),
        ],
        out_specs=pl.BlockSpec((BLK3, DIM), lambda i: (i, 0)),
        out_shape=jax.ShapeDtypeStruct((N, DIM), jnp.float32),
    )(hpart, esum3, bsum3)


def kernel(h, edge_index, sigma_GD, W_fc, b_fc, W_attn, b_attn):
    f32 = jnp.float32
    src = edge_index[0].astype(jnp.int32)
    dst = edge_index[1].astype(jnp.int32)
    sig = sigma_GD.reshape(-1).astype(f32)
    pad_e = EPAD - E
    dummy = N + (jnp.arange(pad_e, dtype=jnp.int32) % (NPAD - N))
    srcp = jnp.concatenate([src, dummy]).reshape(EROWS, 128)
    dstp = jnp.concatenate([dst, dummy]).reshape(EROWS, 128)
    sigp = jnp.concatenate(
        [sig, jnp.zeros((pad_e,), f32)]).reshape(EROWS, 128)
    hp = jnp.pad(h.astype(f32), ((0, NPAD - N), (0, 0)))
    wt = W_fc.T.astype(f32)
    bfc = b_fc.reshape(1, DIM).astype(f32)
    w12 = jnp.stack([W_attn[0, :DIM], W_attn[0, DIM:]], axis=1)
    wa = jnp.pad(w12, ((0, 0), (0, DIM - 2))).astype(f32)
    ba = jnp.zeros((1, DIM), f32).at[0, 1].set(b_attn[0])

    z, s = _project(hp, wt, bfc, wa, ba)
    s1 = s[:, 0]
    s2 = s[:, 1]
    coef, esum_p, bsum_p = _sc_edge_logits(s1, s2, srcp, dstp, sigp)
    edata = jnp.stack(
        [srcp, lax.bitcast_convert_type(coef, jnp.int32)],
        axis=1)
    hpart = _sc_aggregate(z, edata, dstp)
    return _combine(hpart,
                    esum_p.reshape(NC, NPAD, 1),
                    bsum_p.reshape(NC, NPAD, 1))


# speedup vs baseline: 35.0156x; 1.1321x over previous
import functools

import jax
import jax.numpy as jnp
from jax import lax
from jax.experimental import pallas as pl
from jax.experimental.pallas import tpu as pltpu
from jax.experimental.pallas import tpu_sc as plsc

N = 10000
E = 320000
DIM = 128
NC = 2
NS = 16
NW = NC * NS
LANES = 16

NPAD = 10112
NSLICE = NPAD // NS
ROWS_PER_W = 80
EROWS = NW * ROWS_PER_W
EPAD = EROWS * 128

A_CHUNK = 8

_mesh = plsc.VectorSubcoreMesh(
    core_axis_name="c", subcore_axis_name="s", num_cores=NC, num_subcores=NS)


BLK1 = 632

def _project_body(h_ref, wt_ref, b_ref, wa_ref, ba_ref, z_ref, s_ref):
    zb = jnp.dot(h_ref[...], wt_ref[...], preferred_element_type=jnp.float32)
    zb = zb + b_ref[...]
    zb = jnp.where(jnp.abs(zb) == jnp.inf, jnp.float32(1e9), zb)
    z_ref[...] = zb
    sb = jnp.dot(zb, wa_ref[...], preferred_element_type=jnp.float32)
    s_ref[...] = sb + ba_ref[...]


def _project(hp, wt, bfc, wa, ba):
    return pl.pallas_call(
        _project_body,
        grid=(NPAD // BLK1,),
        in_specs=[
            pl.BlockSpec((BLK1, DIM), lambda i: (i, 0)),
            pl.BlockSpec((DIM, DIM), lambda i: (0, 0)),
            pl.BlockSpec((1, DIM), lambda i: (0, 0)),
            pl.BlockSpec((DIM, DIM), lambda i: (0, 0)),
            pl.BlockSpec((1, DIM), lambda i: (0, 0)),
        ],
        out_specs=[
            pl.BlockSpec((BLK1, DIM), lambda i: (i, 0)),
            pl.BlockSpec((BLK1, DIM), lambda i: (i, 0)),
        ],
        out_shape=[
            jax.ShapeDtypeStruct((NPAD, DIM), jnp.float32),
            jax.ShapeDtypeStruct((NPAD, DIM), jnp.float32),
        ],
    )(hp, wt, bfc, wa, ba)


@functools.partial(
    pl.kernel,
    out_type=(
        jax.ShapeDtypeStruct((EROWS, 128), jnp.float32),
        jax.ShapeDtypeStruct((NC * NPAD,), jnp.float32),
        jax.ShapeDtypeStruct((NC * NPAD,), jnp.float32),
    ),
    mesh=_mesh,
    scratch_types=(
        pltpu.VMEM((NPAD,), jnp.float32),
        pltpu.VMEM((NPAD,), jnp.float32),
        pltpu.VMEM((A_CHUNK, 128), jnp.int32),
        pltpu.VMEM((A_CHUNK, 128), jnp.int32),
        pltpu.VMEM((A_CHUNK, 128), jnp.float32),
        pltpu.VMEM((A_CHUNK, 128), jnp.float32),
        pltpu.VMEM((A_CHUNK, 128), jnp.float32),
        pltpu.VMEM((640,), jnp.float32),
        pltpu.VMEM_SHARED((NPAD,), jnp.float32),
        pltpu.VMEM_SHARED((NPAD,), jnp.float32),
    ),
    compiler_params=pltpu.CompilerParams(needs_layout_passes=False),
)
def _sc_edge_logits(s1_hbm, s2_hbm, src_hbm, dst_hbm, sig_hbm,
                    coef_hbm, esum_hbm, bsum_hbm,
                    s1_t, s2_t, src_b, dst_b, sig_b, p_b, coef_b,
                    zb1, esum_sh, bsum_sh):
    cid = lax.axis_index("c")
    sid = lax.axis_index("s")
    wid = cid * NS + sid

    def zloop(i, carry):
        zb1[pl.ds(i * LANES, LANES)] = jnp.zeros((LANES,), jnp.float32)
        return carry
    lax.fori_loop(0, 640 // LANES, zloop, 0)
    base = sid * NSLICE
    pltpu.sync_copy(zb1.at[pl.ds(0, NSLICE)], esum_sh.at[pl.ds(base, NSLICE)])
    pltpu.sync_copy(zb1.at[pl.ds(0, NSLICE)], bsum_sh.at[pl.ds(base, NSLICE)])
    pltpu.sync_copy(s1_hbm, s1_t)
    pltpu.sync_copy(s2_hbm, s2_t)
    plsc.subcore_barrier()

    row0 = wid * ROWS_PER_W
    for ck in range(ROWS_PER_W // A_CHUNK):
        r0 = row0 + ck * A_CHUNK
        pltpu.sync_copy(src_hbm.at[pl.ds(r0, A_CHUNK)], src_b)
        pltpu.sync_copy(dst_hbm.at[pl.ds(r0, A_CHUNK)], dst_b)
        pltpu.sync_copy(sig_hbm.at[pl.ds(r0, A_CHUNK)], sig_b)

        def body(g, carry):
            r = g // (128 // LANES)
            c = (g % (128 // LANES)) * LANES
            si = src_b[r, pl.ds(c, LANES)]
            di = dst_b[r, pl.ds(c, LANES)]
            a = plsc.load_gather(s1_t, [si]) + plsc.load_gather(s2_t, [di])
            a = jnp.where(jnp.abs(a) == jnp.inf, jnp.float32(1e9), a)
            e = jnp.where(a > 0, a, a * jnp.float32(0.01))
            p = jnp.exp(e)
            p_b[r, pl.ds(c, LANES)] = p
            coef_b[r, pl.ds(c, LANES)] = p * sig_b[r, pl.ds(c, LANES)]
            return carry
        lax.fori_loop(0, A_CHUNK * (128 // LANES), body, 0)

        pltpu.sync_copy(coef_b, coef_hbm.at[pl.ds(r0, A_CHUNK)])
        for r in range(A_CHUNK):
            pltpu.sync_copy(p_b.at[r], esum_sh.at[dst_b.at[r]], add=True)
            pltpu.sync_copy(sig_b.at[r], bsum_sh.at[dst_b.at[r]], add=True)

    plsc.subcore_barrier()
    pltpu.sync_copy(esum_sh.at[pl.ds(base, NSLICE)], zb1.at[pl.ds(0, NSLICE)])
    pltpu.sync_copy(zb1.at[pl.ds(0, NSLICE)],
                    esum_hbm.at[pl.ds(cid * NPAD + base, NSLICE)])
    pltpu.sync_copy(bsum_sh.at[pl.ds(base, NSLICE)], zb1.at[pl.ds(0, NSLICE)])
    pltpu.sync_copy(zb1.at[pl.ds(0, NSLICE)],
                    bsum_hbm.at[pl.ds(cid * NPAD + base, NSLICE)])


N_CHUNKS = ROWS_PER_W


@functools.partial(
    pl.kernel,
    out_type=jax.ShapeDtypeStruct((NC, NPAD, DIM), jnp.float32),
    mesh=_mesh,
    scratch_types=(
        pltpu.VMEM((2, 2, 128), jnp.int32),
        pltpu.VMEM((ROWS_PER_W, 128), jnp.int32),
        pltpu.VMEM((128, DIM), jnp.float32),
        pltpu.VMEM((128, DIM), jnp.float32),
        pltpu.VMEM_SHARED((NPAD, DIM), jnp.float32),
        pltpu.SemaphoreType.DMA,
        pltpu.SemaphoreType.DMA,
        pltpu.SemaphoreType.DMA,
    ),
    compiler_params=pltpu.CompilerParams(needs_layout_passes=False),
)
def _sc_aggregate(z_hbm, edata_hbm, dst_hbm, hpart_hbm,
                  ebuf, dst_a, zr0, zr1, hacc_sh, esem, gsem, ssem):
    cid = lax.axis_index("c")
    sid = lax.axis_index("s")
    wid = cid * NS + sid
    zbufs = (zr0, zr1)
    row0 = wid * ROWS_PER_W

    def issue_eload(ck):
        return pltpu.async_copy(edata_hbm.at[row0 + ck], ebuf.at[ck % 2], esem)

    dld = pltpu.async_copy(dst_hbm.at[pl.ds(row0, ROWS_PER_W)], dst_a, esem)
    el = {ck: issue_eload(ck) for ck in range(2)}

    def zloop(i, carry):
        for l in range(DIM // LANES):
            zr0[i, pl.ds(l * LANES, LANES)] = jnp.zeros((LANES,), jnp.float32)
        return carry
    lax.fori_loop(0, 128, zloop, 0)
    base = sid * NSLICE
    off = 0
    while off < NSLICE:
        n = min(128, NSLICE - off)
        pltpu.sync_copy(zr0.at[pl.ds(0, n)], hacc_sh.at[pl.ds(base + off, n)])
        off += n
    plsc.subcore_barrier()
    dld.wait()

    def issue_gather(ck):
        return pltpu.async_copy(z_hbm.at[ebuf.at[ck % 2, 0]],
                                zbufs[ck % 2], gsem)

    def issue_scatter(ck):
        return pltpu.async_copy(zbufs[ck % 2], hacc_sh.at[dst_a.at[ck]],
                                ssem, add=True)

    def scale(ck):
        buf = zbufs[ck % 2]
        s = ck % 2

        @plsc.parallel_loop(0, 128, unroll=2)
        def _(j):
            cbits = plsc.load_gather(
                ebuf, [jnp.full((LANES,), s, jnp.int32),
                       jnp.full((LANES,), 1, jnp.int32),
                       jnp.full((LANES,), j, jnp.int32)])
            cval = plsc.bitcast(cbits, jnp.float32)
            for l in range(DIM // LANES):
                buf[j, pl.ds(l * LANES, LANES)] = (
                    buf[j, pl.ds(l * LANES, LANES)] * cval)

    el.pop(0).wait()
    gd = {0: issue_gather(0)}
    sd = {}
    for ck in range(N_CHUNKS):
        if ck >= 1:
            sd.pop(ck - 1).wait()
        if ck + 1 < N_CHUNKS:
            el.pop(ck + 1).wait()
            gd[ck + 1] = issue_gather(ck + 1)
        gd.pop(ck).wait()
        scale(ck)
        sd[ck] = issue_scatter(ck)
        if ck + 2 < N_CHUNKS:
            el[ck + 2] = issue_eload(ck + 2)
    sd.pop(N_CHUNKS - 1).wait()

    plsc.subcore_barrier()
    off = 0
    while off < NSLICE:
        n = min(128, NSLICE - off)
        pltpu.sync_copy(hacc_sh.at[pl.ds(base + off, n)], zr0.at[pl.ds(0, n)])
        pltpu.sync_copy(zr0.at[pl.ds(0, n)],
                        hpart_hbm.at[cid, pl.ds(base + off, n)])
        off += n


BLK3 = 400

def _combine_body(hp_ref, es_ref, bs_ref, o_ref):
    hp = hp_ref[0] + hp_ref[1]
    es = es_ref[0] + es_ref[1]
    bs = bs_ref[0] + bs_ref[1]
    den = es * (bs + jnp.float32(1e-6))
    t = jnp.where(es > 0, hp / den, jnp.float32(0.0))
    t = jnp.where(jnp.abs(t) == jnp.inf, jnp.float32(1e9), t)
    o_ref[...] = jnp.where(t > 0, t, jnp.exp(t) - jnp.float32(1.0))


def _combine(hpart, esum3, bsum3):
    return pl.pallas_call(
        _combine_body,
        grid=(N // BLK3,),
        in_specs=[
            pl.BlockSpec((NC, BLK3, DIM), lambda i: (0, i, 0)),
            pl.BlockSpec((NC, BLK3, 1), lambda i: (0, i, 0)),
            pl.BlockSpec((NC, BLK3, 1), lambda i: (0, i, 0)),
        ],
        out_specs=pl.BlockSpec((BLK3, DIM), lambda i: (i, 0)),
        out_shape=jax.ShapeDtypeStruct((N, DIM), jnp.float32),
    )(hpart, esum3, bsum3)


def kernel(h, edge_index, sigma_GD, W_fc, b_fc, W_attn, b_attn):
    f32 = jnp.float32
    src = edge_index[0].astype(jnp.int32)
    dst = edge_index[1].astype(jnp.int32)
    sig = sigma_GD.reshape(-1).astype(f32)
    pad_e = EPAD - E
    dummy = N + (jnp.arange(pad_e, dtype=jnp.int32) % (NPAD - N))
    srcp = jnp.concatenate([src, dummy]).reshape(EROWS, 128)
    dstp = jnp.concatenate([dst, dummy]).reshape(EROWS, 128)
    sigp = jnp.concatenate(
        [sig, jnp.zeros((pad_e,), f32)]).reshape(EROWS, 128)
    hp = jnp.pad(h.astype(f32), ((0, NPAD - N), (0, 0)))
    wt = W_fc.T.astype(f32)
    bfc = b_fc.reshape(1, DIM).astype(f32)
    w12 = jnp.stack([W_attn[0, :DIM], W_attn[0, DIM:]], axis=1)
    wa = jnp.pad(w12, ((0, 0), (0, DIM - 2))).astype(f32)
    ba = jnp.zeros((1, DIM), f32).at[0, 1].set(b_attn[0])

    z, s = _project(hp, wt, bfc, wa, ba)
    s1 = s[:, 0]
    s2 = s[:, 1]
    coef, esum_p, bsum_p = _sc_edge_logits(s1, s2, srcp, dstp, sigp)
    edata = jnp.stack(
        [srcp, lax.bitcast_convert_type(coef, jnp.int32)],
        axis=1)
    hpart = _sc_aggregate(z, edata, dstp)
    return _combine(hpart,
                    esum_p.reshape(NC, NPAD, 1),
                    bsum_p.reshape(NC, NPAD, 1))


# speedup vs baseline: 40.0794x; 1.1446x over previous
import functools

import jax
import jax.numpy as jnp
from jax import lax
from jax.experimental import pallas as pl
from jax.experimental.pallas import tpu as pltpu
from jax.experimental.pallas import tpu_sc as plsc

N = 10000
E = 320000
DIM = 128
NC = 2
NS = 16
NW = NC * NS
LANES = 16

NPAD = 10112
NSLICE = NPAD // NS
ROWS_PER_W = 80
EROWS = NW * ROWS_PER_W
EPAD = EROWS * 128

A_CHUNK = 8

_mesh = plsc.VectorSubcoreMesh(
    core_axis_name="c", subcore_axis_name="s", num_cores=NC, num_subcores=NS)


BLK1 = 632

def _project_body(h_ref, wt_ref, b_ref, wa_ref, ba_ref, z_ref, s_ref):
    zb = jnp.dot(h_ref[...], wt_ref[...], preferred_element_type=jnp.float32)
    zb = zb + b_ref[...]
    zb = jnp.where(jnp.abs(zb) == jnp.inf, jnp.float32(1e9), zb)
    z_ref[...] = zb
    sb = jnp.dot(zb, wa_ref[...], preferred_element_type=jnp.float32)
    s_ref[...] = sb + ba_ref[...]


def _project(hp, wt, bfc, wa, ba):
    return pl.pallas_call(
        _project_body,
        grid=(NPAD // BLK1,),
        in_specs=[
            pl.BlockSpec((BLK1, DIM), lambda i: (i, 0)),
            pl.BlockSpec((DIM, DIM), lambda i: (0, 0)),
            pl.BlockSpec((1, DIM), lambda i: (0, 0)),
            pl.BlockSpec((DIM, DIM), lambda i: (0, 0)),
            pl.BlockSpec((1, DIM), lambda i: (0, 0)),
        ],
        out_specs=[
            pl.BlockSpec((BLK1, DIM), lambda i: (i, 0)),
            pl.BlockSpec((BLK1, DIM), lambda i: (i, 0)),
        ],
        out_shape=[
            jax.ShapeDtypeStruct((NPAD, DIM), jnp.float32),
            jax.ShapeDtypeStruct((NPAD, DIM), jnp.float32),
        ],
    )(hp, wt, bfc, wa, ba)


@functools.partial(
    pl.kernel,
    out_type=(
        jax.ShapeDtypeStruct((EROWS, 128), jnp.float32),
        jax.ShapeDtypeStruct((NC * NPAD,), jnp.float32),
        jax.ShapeDtypeStruct((NC * NPAD,), jnp.float32),
    ),
    mesh=_mesh,
    scratch_types=(
        pltpu.VMEM((NPAD,), jnp.float32),
        pltpu.VMEM((NPAD,), jnp.float32),
        pltpu.VMEM((ROWS_PER_W, 128), jnp.int32),
        pltpu.VMEM((ROWS_PER_W, 128), jnp.int32),
        pltpu.VMEM((ROWS_PER_W, 128), jnp.float32),
        pltpu.VMEM((ROWS_PER_W, 128), jnp.float32),
        pltpu.VMEM((ROWS_PER_W, 128), jnp.float32),
        pltpu.VMEM((640,), jnp.float32),
        pltpu.VMEM_SHARED((NPAD,), jnp.float32),
        pltpu.VMEM_SHARED((NPAD,), jnp.float32),
        pltpu.SemaphoreType.DMA,
        pltpu.SemaphoreType.DMA,
    ),
    compiler_params=pltpu.CompilerParams(needs_layout_passes=False),
)
def _sc_edge_logits(s1_hbm, s2_hbm, src_hbm, dst_hbm, sig_hbm,
                    coef_hbm, esum_hbm, bsum_hbm,
                    s1_t, s2_t, src_a, dst_a, sig_a, p_a, coef_a,
                    zb1, esum_sh, bsum_sh, lsem, ssem):
    cid = lax.axis_index("c")
    sid = lax.axis_index("s")
    wid = cid * NS + sid
    row0 = wid * ROWS_PER_W

    loads = [
        pltpu.async_copy(src_hbm.at[pl.ds(row0, ROWS_PER_W)], src_a, lsem),
        pltpu.async_copy(dst_hbm.at[pl.ds(row0, ROWS_PER_W)], dst_a, lsem),
        pltpu.async_copy(sig_hbm.at[pl.ds(row0, ROWS_PER_W)], sig_a, lsem),
        pltpu.async_copy(s1_hbm, s1_t, lsem),
        pltpu.async_copy(s2_hbm, s2_t, lsem),
    ]

    def zloop(i, carry):
        zb1[pl.ds(i * LANES, LANES)] = jnp.zeros((LANES,), jnp.float32)
        return carry
    lax.fori_loop(0, 640 // LANES, zloop, 0)
    base = sid * NSLICE
    pltpu.sync_copy(zb1.at[pl.ds(0, NSLICE)], esum_sh.at[pl.ds(base, NSLICE)])
    pltpu.sync_copy(zb1.at[pl.ds(0, NSLICE)], bsum_sh.at[pl.ds(base, NSLICE)])
    plsc.subcore_barrier()
    for d in loads:
        d.wait()

    @plsc.parallel_loop(0, ROWS_PER_W * (128 // LANES), unroll=2)
    def _(g):
        r = g // (128 // LANES)
        c = (g % (128 // LANES)) * LANES
        si = src_a[r, pl.ds(c, LANES)]
        di = dst_a[r, pl.ds(c, LANES)]
        a = plsc.load_gather(s1_t, [si]) + plsc.load_gather(s2_t, [di])
        a = jnp.where(jnp.abs(a) == jnp.inf, jnp.float32(1e9), a)
        e = jnp.where(a > 0, a, a * jnp.float32(0.01))
        p = jnp.exp(e)
        p_a[r, pl.ds(c, LANES)] = p
        coef_a[r, pl.ds(c, LANES)] = p * sig_a[r, pl.ds(c, LANES)]

    cpy = pltpu.async_copy(coef_a, coef_hbm.at[pl.ds(row0, ROWS_PER_W)], lsem)
    scat = []
    for r in range(ROWS_PER_W):
        scat.append(pltpu.async_copy(
            p_a.at[r], esum_sh.at[dst_a.at[r]], ssem, add=True))
        scat.append(pltpu.async_copy(
            sig_a.at[r], bsum_sh.at[dst_a.at[r]], ssem, add=True))
    cpy.wait()
    for d in scat:
        d.wait()

    plsc.subcore_barrier()
    pltpu.sync_copy(esum_sh.at[pl.ds(base, NSLICE)], zb1.at[pl.ds(0, NSLICE)])
    pltpu.sync_copy(zb1.at[pl.ds(0, NSLICE)],
                    esum_hbm.at[pl.ds(cid * NPAD + base, NSLICE)])
    pltpu.sync_copy(bsum_sh.at[pl.ds(base, NSLICE)], zb1.at[pl.ds(0, NSLICE)])
    pltpu.sync_copy(zb1.at[pl.ds(0, NSLICE)],
                    bsum_hbm.at[pl.ds(cid * NPAD + base, NSLICE)])


N_CHUNKS = ROWS_PER_W


@functools.partial(
    pl.kernel,
    out_type=jax.ShapeDtypeStruct((NC, NPAD, DIM), jnp.float32),
    mesh=_mesh,
    scratch_types=(
        pltpu.VMEM((2, 2, 128), jnp.int32),
        pltpu.VMEM((ROWS_PER_W, 128), jnp.int32),
        pltpu.VMEM((128, DIM), jnp.float32),
        pltpu.VMEM((128, DIM), jnp.float32),
        pltpu.VMEM_SHARED((NPAD, DIM), jnp.float32),
        pltpu.SemaphoreType.DMA,
        pltpu.SemaphoreType.DMA,
        pltpu.SemaphoreType.DMA,
    ),
    compiler_params=pltpu.CompilerParams(needs_layout_passes=False),
)
def _sc_aggregate(z_hbm, edata_hbm, dst_hbm, hpart_hbm,
                  ebuf, dst_a, zr0, zr1, hacc_sh, esem, gsem, ssem):
    cid = lax.axis_index("c")
    sid = lax.axis_index("s")
    wid = cid * NS + sid
    zbufs = (zr0, zr1)
    row0 = wid * ROWS_PER_W

    def issue_eload(ck):
        return pltpu.async_copy(edata_hbm.at[row0 + ck], ebuf.at[ck % 2], esem)

    dld = pltpu.async_copy(dst_hbm.at[pl.ds(row0, ROWS_PER_W)], dst_a, esem)
    el = {ck: issue_eload(ck) for ck in range(2)}

    def zloop(i, carry):
        for l in range(DIM // LANES):
            zr0[i, pl.ds(l * LANES, LANES)] = jnp.zeros((LANES,), jnp.float32)
        return carry
    lax.fori_loop(0, 128, zloop, 0)
    base = sid * NSLICE
    off = 0
    while off < NSLICE:
        n = min(128, NSLICE - off)
        pltpu.sync_copy(zr0.at[pl.ds(0, n)], hacc_sh.at[pl.ds(base + off, n)])
        off += n
    plsc.subcore_barrier()
    dld.wait()

    def issue_gather(ck):
        return pltpu.async_copy(z_hbm.at[ebuf.at[ck % 2, 0]],
                                zbufs[ck % 2], gsem)

    def issue_scatter(ck):
        return pltpu.async_copy(zbufs[ck % 2], hacc_sh.at[dst_a.at[ck]],
                                ssem, add=True)

    def scale(ck):
        buf = zbufs[ck % 2]
        s = ck % 2

        @plsc.parallel_loop(0, 128, unroll=2)
        def _(j):
            cbits = plsc.load_gather(
                ebuf, [jnp.full((LANES,), s, jnp.int32),
                       jnp.full((LANES,), 1, jnp.int32),
                       jnp.full((LANES,), j, jnp.int32)])
            cval = plsc.bitcast(cbits, jnp.float32)
            for l in range(DIM // LANES):
                buf[j, pl.ds(l * LANES, LANES)] = (
                    buf[j, pl.ds(l * LANES, LANES)] * cval)

    el.pop(0).wait()
    gd = {0: issue_gather(0)}
    sd = {}
    for ck in range(N_CHUNKS):
        if ck >= 1:
            sd.pop(ck - 1).wait()
        if ck + 1 < N_CHUNKS:
            el.pop(ck + 1).wait()
            gd[ck + 1] = issue_gather(ck + 1)
        gd.pop(ck).wait()
        scale(ck)
        sd[ck] = issue_scatter(ck)
        if ck + 2 < N_CHUNKS:
            el[ck + 2] = issue_eload(ck + 2)
    sd.pop(N_CHUNKS - 1).wait()

    plsc.subcore_barrier()
    off = 0
    while off < NSLICE:
        n = min(128, NSLICE - off)
        pltpu.sync_copy(hacc_sh.at[pl.ds(base + off, n)], zr0.at[pl.ds(0, n)])
        pltpu.sync_copy(zr0.at[pl.ds(0, n)],
                        hpart_hbm.at[cid, pl.ds(base + off, n)])
        off += n


BLK3 = 400

def _combine_body(hp_ref, es_ref, bs_ref, o_ref):
    hp = hp_ref[0] + hp_ref[1]
    es = es_ref[0] + es_ref[1]
    bs = bs_ref[0] + bs_ref[1]
    den = es * (bs + jnp.float32(1e-6))
    t = jnp.where(es > 0, hp / den, jnp.float32(0.0))
    t = jnp.where(jnp.abs(t) == jnp.inf, jnp.float32(1e9), t)
    o_ref[...] = jnp.where(t > 0, t, jnp.exp(t) - jnp.float32(1.0))


def _combine(hpart, esum3, bsum3):
    return pl.pallas_call(
        _combine_body,
        grid=(N // BLK3,),
        in_specs=[
            pl.BlockSpec((NC, BLK3, DIM), lambda i: (0, i, 0)),
            pl.BlockSpec((NC, BLK3, 1), lambda i: (0, i, 0)),
            pl.BlockSpec((NC, BLK3, 1), lambda i: (0, i, 0)),
        ],
        out_specs=pl.BlockSpec((BLK3, DIM), lambda i: (i, 0)),
        out_shape=jax.ShapeDtypeStruct((N, DIM), jnp.float32),
    )(hpart, esum3, bsum3)


def kernel(h, edge_index, sigma_GD, W_fc, b_fc, W_attn, b_attn):
    f32 = jnp.float32
    src = edge_index[0].astype(jnp.int32)
    dst = edge_index[1].astype(jnp.int32)
    sig = sigma_GD.reshape(-1).astype(f32)
    pad_e = EPAD - E
    dummy = N + (jnp.arange(pad_e, dtype=jnp.int32) % (NPAD - N))
    srcp = jnp.concatenate([src, dummy]).reshape(EROWS, 128)
    dstp = jnp.concatenate([dst, dummy]).reshape(EROWS, 128)
    sigp = jnp.concatenate(
        [sig, jnp.zeros((pad_e,), f32)]).reshape(EROWS, 128)
    hp = jnp.pad(h.astype(f32), ((0, NPAD - N), (0, 0)))
    wt = W_fc.T.astype(f32)
    bfc = b_fc.reshape(1, DIM).astype(f32)
    w12 = jnp.stack([W_attn[0, :DIM], W_attn[0, DIM:]], axis=1)
    wa = jnp.pad(w12, ((0, 0), (0, DIM - 2))).astype(f32)
    ba = jnp.zeros((1, DIM), f32).at[0, 1].set(b_attn[0])

    z, s = _project(hp, wt, bfc, wa, ba)
    s1 = s[:, 0]
    s2 = s[:, 1]
    coef, esum_p, bsum_p = _sc_edge_logits(s1, s2, srcp, dstp, sigp)
    edata = jnp.stack(
        [srcp, lax.bitcast_convert_type(coef, jnp.int32)],
        axis=1)
    hpart = _sc_aggregate(z, edata, dstp)
    return _combine(hpart,
                    esum_p.reshape(NC, NPAD, 1),
                    bsum_p.reshape(NC, NPAD, 1))
